# baseline (speedup 1.0000x reference)
import functools

import jax
import jax.numpy as jnp
from jax import lax
from jax.experimental import pallas as pl
from jax.experimental.pallas import tpu as pltpu
from jax.experimental.pallas import tpu_sc as plsc

_S = 20000
_SPAD = 20480
_D = 256
_R = 1024
_NT = _SPAD // _R
_BUDGET = 64
_NEG = -1e9
_PHASES = 1

_NC = 2
_NW = 32


@functools.cache
def _sc_gather(nrows, ch):
    bpw = nrows // _NW
    nch = bpw // ch

    @functools.partial(
        pl.kernel,
        mesh=plsc.VectorSubcoreMesh(core_axis_name="c", subcore_axis_name="s"),
        out_type=jax.ShapeDtypeStruct((nrows, _D), jnp.float32),
        scratch_types=[
            pltpu.VMEM((nch, ch), jnp.int32),
            pltpu.VMEM((2, ch, _D), jnp.float32),
            pltpu.SemaphoreType.DMA,
            pltpu.SemaphoreType.DMA,
        ],
    )
    def gather(idx_hbm, table_hbm, out_hbm, idx_v, rows_v, sem0, sem1):
        wid = lax.axis_index("s") * _NC + lax.axis_index("c")
        base = wid * bpw
        pltpu.sync_copy(idx_hbm.at[wid], idx_v)
        sems = (sem0, sem1)
        cps = [None, None]
        cps[0] = pltpu.async_copy(table_hbm.at[idx_v.at[0]], rows_v.at[0], sem0)
        for t in range(nch):
            if t + 1 < nch:
                b = (t + 1) % 2
                cps[b] = pltpu.async_copy(
                    table_hbm.at[idx_v.at[t + 1]], rows_v.at[b], sems[b])
            cps[t % 2].wait()
            pltpu.sync_copy(rows_v.at[t % 2],
                            out_hbm.at[pl.ds(base + t * ch, ch)])

    return gather


def _score_body(flat_base, g_ref, adj_ref, tar_ref, add_ref, wlabel_ref,
                wsec_ref, w1_ref, w2_ref, l1wt_ref, l1b_ref, l2wt_ref,
                l2b_ref, l3w_ref, l3b_ref, out_ref):
    f32 = jnp.float32
    g = g_ref[...]
    sub = jnp.dot(jnp.dot(g, w1_ref[...], preferred_element_type=f32),
                  w2_ref[...], preferred_element_type=f32)
    concat = jnp.concatenate([
        jnp.broadcast_to(tar_ref[...], (_R, 64)),
        sub,
        jnp.broadcast_to(add_ref[...], (_R, 64)),
        adj_ref[...],
        jnp.broadcast_to(wlabel_ref[...], (_R, 256)),
        jnp.broadcast_to(wsec_ref[...], (_R, 256)),
    ], axis=1)
    h = jnp.dot(concat, l1wt_ref[...], preferred_element_type=f32) + l1b_ref[...]
    h = jnp.where(h >= 0, h, 0.01 * h)
    h = jnp.dot(h, l2wt_ref[...], preferred_element_type=f32) + l2b_ref[...]
    h = jnp.where(h >= 0, h, 0.01 * h)
    sc = lax.dot_general(l3w_ref[...], h, (((1,), (1,)), ((), ())),
                         preferred_element_type=f32) + l3b_ref[...]
    i = pl.program_id(0)
    flat = flat_base + i * _R + lax.broadcasted_iota(jnp.int32, (1, _R), 1)
    out_ref[...] = jnp.where(flat < _S, sc, _NEG).reshape(_R // 128, 128)


def _topk_body(score_ref, disc_ref, u_ref, e_ref):
    rows = _SPAD // 128
    flat = (lax.broadcasted_iota(jnp.int32, (rows, 128), 0) * 128
            + lax.broadcasted_iota(jnp.int32, (rows, 128), 1))
    big = jnp.int32(2**30)
    disc_ref[...] = jnp.zeros((rows, 128), jnp.float32)
    u_ref[...] = score_ref[...] / 0.01
    e_ref[...] = jnp.exp(u_ref[...] - jnp.max(u_ref[...]))

    def it(t, c):
        emax = jnp.max(e_ref[...])
        do_rescale = emax < 1e-20

        @pl.when(do_rescale)
        def _():
            disc_ref[...] = disc_ref[...] + e_ref[...] * c
            e_ref[...] = jnp.exp(u_ref[...] - jnp.max(u_ref[...]))

        c2 = jnp.where(do_rescale, jnp.float32(0.0), c)
        emax2 = jnp.where(do_rescale, jnp.float32(1.0), emax)
        c3 = c2 + 1.0 / jnp.sum(e_ref[...])
        sel = jnp.min(jnp.where(e_ref[...] == emax2, flat, big))
        issel = flat == sel
        disc_ref[...] = disc_ref[...] + jnp.where(issel, emax2 * c3, 0.0)
        e_ref[...] = jnp.where(issel, jnp.float32(0.0), e_ref[...])
        u_ref[...] = jnp.where(issel, jnp.float32(-1e30), u_ref[...])
        return c3

    c_end = lax.fori_loop(0, _BUDGET, it, jnp.float32(0.0))
    disc_ref[...] = disc_ref[...] + e_ref[...] * c_end


def _score_call(g, adj_pad, tar_xw, add_xw, wlabel, wsec,
                w1, w2, l1wt, l1b, l2wt, l2b, l3w, l3b,
                flat_base=0, interp=False):
    const = lambda i: (0, 0)
    nrows = g.shape[0]
    return pl.pallas_call(
        functools.partial(_score_body, flat_base),
        grid=(nrows // _R,),
        in_specs=[
            pl.BlockSpec((_R, _D), lambda i: (i, 0)),
            pl.BlockSpec((_R, 1), lambda i: (i, 0)),
            pl.BlockSpec((1, 64), const),
            pl.BlockSpec((1, 64), const),
            pl.BlockSpec((1, 256), const),
            pl.BlockSpec((1, 256), const),
            pl.BlockSpec((256, 128), const),
            pl.BlockSpec((128, 64), const),
            pl.BlockSpec((705, 512), const),
            pl.BlockSpec((1, 512), const),
            pl.BlockSpec((512, 32), const),
            pl.BlockSpec((1, 32), const),
            pl.BlockSpec((1, 32), const),
            pl.BlockSpec((1, 1), const),
        ],
        out_specs=pl.BlockSpec((_R // 128, 128), lambda i: (i, 0)),
        out_shape=jax.ShapeDtypeStruct((nrows // 128, 128), jnp.float32),
        interpret=interp,
    )(g, adj_pad, tar_xw, add_xw, wlabel, wsec, w1, w2, l1wt, l1b, l2wt, l2b,
      l3w, l3b)


def _topk_call(score2d, interp=False):
    return pl.pallas_call(
        _topk_body,
        out_shape=jax.ShapeDtypeStruct((_SPAD // 128, 128), jnp.float32),
        scratch_shapes=[pltpu.VMEM((_SPAD // 128, 128), jnp.float32),
                        pltpu.VMEM((_SPAD // 128, 128), jnp.float32)],
        interpret=interp,
    )(score2d)


def kernel(budget, target, sub_graph_nodes, new_feat, adj_tensor, wlabel, wsec,
           weight1, weight2, l1_w, l1_b, l2_w, l2_b, l3_w, l3_b):
    f32 = jnp.float32
    tar_xw = (jnp.take(new_feat, target, axis=0) @ weight1) @ weight2
    add_xw = (new_feat[-1:, :] @ weight1) @ weight2

    idx_pad = jnp.concatenate(
        [sub_graph_nodes.astype(jnp.int32), jnp.zeros((_SPAD - _S,), jnp.int32)])
    adj_pad = jnp.concatenate(
        [adj_tensor, jnp.zeros((_SPAD - _S, 1), f32)], axis=0)

    pr = _SPAD // _PHASES
    ch = 80
    weights = (tar_xw, add_xw, wlabel, wsec, weight1, weight2,
               l1_w.T, l1_b[None, :], l2_w.T, l2_b[None, :],
               l3_w, l3_b[None, :])
    gs = [_sc_gather(pr, ch)(
        idx_pad[p * pr:(p + 1) * pr].reshape(_NW, -1, ch), new_feat)
        for p in range(_PHASES)]
    ss = [_score_call(gs[p], adj_pad[p * pr:(p + 1) * pr], *weights,
                      flat_base=p * pr) for p in range(_PHASES)]
    disc2d = _topk_call(jnp.concatenate(ss, axis=0) if _PHASES > 1 else ss[0])
    discrete = disc2d.reshape(-1)[:_S]
    return (discrete, sub_graph_nodes[None, :])


# speedup vs baseline: 1.0510x; 1.0510x over previous
import functools

import jax
import jax.numpy as jnp
from jax import lax
from jax.experimental import pallas as pl
from jax.experimental.pallas import tpu as pltpu
from jax.experimental.pallas import tpu_sc as plsc

_S = 20000
_SPAD = 20480
_D = 256
_R = 2048
_NT = _SPAD // _R
_BUDGET = 64
_NEG = -1e9
_PHASES = 2

_NC = 2
_NW = 32


@functools.cache
def _sc_gather(nrows, ch):
    bpw = nrows // _NW
    nch = bpw // ch

    @functools.partial(
        pl.kernel,
        mesh=plsc.VectorSubcoreMesh(core_axis_name="c", subcore_axis_name="s"),
        out_type=jax.ShapeDtypeStruct((nrows, _D), jnp.float32),
        scratch_types=[
            pltpu.VMEM((nch, ch), jnp.int32),
            pltpu.VMEM((2, ch, _D), jnp.float32),
            pltpu.SemaphoreType.DMA,
            pltpu.SemaphoreType.DMA,
        ],
    )
    def gather(idx_hbm, table_hbm, out_hbm, idx_v, rows_v, sem0, sem1):
        wid = lax.axis_index("s") * _NC + lax.axis_index("c")
        base = wid * bpw
        pltpu.sync_copy(idx_hbm.at[wid], idx_v)
        sems = (sem0, sem1)
        cps = [None, None]
        cps[0] = pltpu.async_copy(table_hbm.at[idx_v.at[0]], rows_v.at[0], sem0)
        for t in range(nch):
            if t + 1 < nch:
                b = (t + 1) % 2
                cps[b] = pltpu.async_copy(
                    table_hbm.at[idx_v.at[t + 1]], rows_v.at[b], sems[b])
            cps[t % 2].wait()
            pltpu.sync_copy(rows_v.at[t % 2],
                            out_hbm.at[pl.ds(base + t * ch, ch)])

    return gather


def _score_body(flat_base, g_ref, adj_ref, tar_ref, add_ref, wlabel_ref,
                wsec_ref, w1_ref, w2_ref, l1wt_ref, l1b_ref, l2wt_ref,
                l2b_ref, l3w_ref, l3b_ref, out_ref):
    f32 = jnp.float32
    g = g_ref[...]
    sub = jnp.dot(jnp.dot(g, w1_ref[...], preferred_element_type=f32),
                  w2_ref[...], preferred_element_type=f32)
    concat = jnp.concatenate([
        jnp.broadcast_to(tar_ref[...], (_R, 64)),
        sub,
        jnp.broadcast_to(add_ref[...], (_R, 64)),
        adj_ref[...],
        jnp.broadcast_to(wlabel_ref[...], (_R, 256)),
        jnp.broadcast_to(wsec_ref[...], (_R, 256)),
    ], axis=1)
    h = jnp.dot(concat, l1wt_ref[...], preferred_element_type=f32) + l1b_ref[...]
    h = jnp.where(h >= 0, h, 0.01 * h)
    h = jnp.dot(h, l2wt_ref[...], preferred_element_type=f32) + l2b_ref[...]
    h = jnp.where(h >= 0, h, 0.01 * h)
    sc = lax.dot_general(l3w_ref[...], h, (((1,), (1,)), ((), ())),
                         preferred_element_type=f32) + l3b_ref[...]
    i = pl.program_id(0)
    flat = flat_base + i * _R + lax.broadcasted_iota(jnp.int32, (1, _R), 1)
    out_ref[...] = jnp.where(flat < _S, sc, _NEG).reshape(_R // 128, 128)


def _topk_body(score_ref, disc_ref, u_ref, e_ref):
    rows = _SPAD // 128
    flat = (lax.broadcasted_iota(jnp.int32, (rows, 128), 0) * 128
            + lax.broadcasted_iota(jnp.int32, (rows, 128), 1))
    big = jnp.int32(2**30)
    disc_ref[...] = jnp.zeros((rows, 128), jnp.float32)
    u_ref[...] = score_ref[...] / 0.01
    e_ref[...] = jnp.exp(u_ref[...] - jnp.max(u_ref[...]))

    def it(t, c):
        emax = jnp.max(e_ref[...])
        do_rescale = emax < 1e-20

        @pl.when(do_rescale)
        def _():
            disc_ref[...] = disc_ref[...] + e_ref[...] * c
            e_ref[...] = jnp.exp(u_ref[...] - jnp.max(u_ref[...]))

        c2 = jnp.where(do_rescale, jnp.float32(0.0), c)
        emax2 = jnp.where(do_rescale, jnp.float32(1.0), emax)
        c3 = c2 + 1.0 / jnp.sum(e_ref[...])
        sel = jnp.min(jnp.where(e_ref[...] == emax2, flat, big))
        issel = flat == sel
        disc_ref[...] = disc_ref[...] + jnp.where(issel, emax2 * c3, 0.0)
        e_ref[...] = jnp.where(issel, jnp.float32(0.0), e_ref[...])
        u_ref[...] = jnp.where(issel, jnp.float32(-1e30), u_ref[...])
        return c3

    c_end = lax.fori_loop(0, _BUDGET, it, jnp.float32(0.0))
    disc_ref[...] = disc_ref[...] + e_ref[...] * c_end


def _score_call(g, adj_pad, tar_xw, add_xw, wlabel, wsec,
                w1, w2, l1wt, l1b, l2wt, l2b, l3w, l3b,
                flat_base=0, interp=False):
    const = lambda i: (0, 0)
    nrows = g.shape[0]
    return pl.pallas_call(
        functools.partial(_score_body, flat_base),
        grid=(nrows // _R,),
        in_specs=[
            pl.BlockSpec((_R, _D), lambda i: (i, 0)),
            pl.BlockSpec((_R, 1), lambda i: (i, 0)),
            pl.BlockSpec((1, 64), const),
            pl.BlockSpec((1, 64), const),
            pl.BlockSpec((1, 256), const),
            pl.BlockSpec((1, 256), const),
            pl.BlockSpec((256, 128), const),
            pl.BlockSpec((128, 64), const),
            pl.BlockSpec((705, 512), const),
            pl.BlockSpec((1, 512), const),
            pl.BlockSpec((512, 32), const),
            pl.BlockSpec((1, 32), const),
            pl.BlockSpec((1, 32), const),
            pl.BlockSpec((1, 1), const),
        ],
        out_specs=pl.BlockSpec((_R // 128, 128), lambda i: (i, 0)),
        out_shape=jax.ShapeDtypeStruct((nrows // 128, 128), jnp.float32),
        interpret=interp,
    )(g, adj_pad, tar_xw, add_xw, wlabel, wsec, w1, w2, l1wt, l1b, l2wt, l2b,
      l3w, l3b)


def _topk_call(score2d, interp=False):
    return pl.pallas_call(
        _topk_body,
        out_shape=jax.ShapeDtypeStruct((_SPAD // 128, 128), jnp.float32),
        scratch_shapes=[pltpu.VMEM((_SPAD // 128, 128), jnp.float32),
                        pltpu.VMEM((_SPAD // 128, 128), jnp.float32)],
        interpret=interp,
    )(score2d)


def kernel(budget, target, sub_graph_nodes, new_feat, adj_tensor, wlabel, wsec,
           weight1, weight2, l1_w, l1_b, l2_w, l2_b, l3_w, l3_b):
    f32 = jnp.float32
    tar_xw = (jnp.take(new_feat, target, axis=0) @ weight1) @ weight2
    add_xw = (new_feat[-1:, :] @ weight1) @ weight2

    idx_pad = jnp.concatenate(
        [sub_graph_nodes.astype(jnp.int32), jnp.zeros((_SPAD - _S,), jnp.int32)])
    adj_pad = jnp.concatenate(
        [adj_tensor, jnp.zeros((_SPAD - _S, 1), f32)], axis=0)

    pr = _SPAD // _PHASES
    ch = 80
    weights = (tar_xw, add_xw, wlabel, wsec, weight1, weight2,
               l1_w.T, l1_b[None, :], l2_w.T, l2_b[None, :],
               l3_w, l3_b[None, :])
    gs = [_sc_gather(pr, ch)(
        idx_pad[p * pr:(p + 1) * pr].reshape(_NW, -1, ch), new_feat)
        for p in range(_PHASES)]
    ss = [_score_call(gs[p], adj_pad[p * pr:(p + 1) * pr], *weights,
                      flat_base=p * pr) for p in range(_PHASES)]
    disc2d = _topk_call(jnp.concatenate(ss, axis=0) if _PHASES > 1 else ss[0])
    discrete = disc2d.reshape(-1)[:_S]
    return (discrete, sub_graph_nodes[None, :])


# speedup vs baseline: 1.0602x; 1.0088x over previous
import functools

import jax
import jax.numpy as jnp
from jax import lax
from jax.experimental import pallas as pl
from jax.experimental.pallas import tpu as pltpu
from jax.experimental.pallas import tpu_sc as plsc

_S = 20000
_SPAD = 20480
_D = 256
_R = 2048
_NT = _SPAD // _R
_BUDGET = 64
_NEG = -1e9
_PHASES = 2

_NC = 2
_NW = 32


@functools.cache
def _sc_gather(nrows, ch):
    bpw = nrows // _NW
    nch = bpw // ch

    @functools.partial(
        pl.kernel,
        mesh=plsc.VectorSubcoreMesh(core_axis_name="c", subcore_axis_name="s"),
        out_type=jax.ShapeDtypeStruct((nrows, _D), jnp.float32),
        scratch_types=[
            pltpu.VMEM((nch, ch), jnp.int32),
            pltpu.VMEM((2, ch, _D), jnp.float32),
            pltpu.SemaphoreType.DMA,
            pltpu.SemaphoreType.DMA,
        ],
    )
    def gather(idx_hbm, table_hbm, out_hbm, idx_v, rows_v, sem0, sem1):
        wid = lax.axis_index("s") * _NC + lax.axis_index("c")
        base = wid * bpw
        pltpu.sync_copy(idx_hbm.at[wid], idx_v)
        sems = (sem0, sem1)
        cps = [None, None]
        cps[0] = pltpu.async_copy(table_hbm.at[idx_v.at[0]], rows_v.at[0], sem0)
        for t in range(nch):
            if t + 1 < nch:
                b = (t + 1) % 2
                cps[b] = pltpu.async_copy(
                    table_hbm.at[idx_v.at[t + 1]], rows_v.at[b], sems[b])
            cps[t % 2].wait()
            pltpu.sync_copy(rows_v.at[t % 2],
                            out_hbm.at[pl.ds(base + t * ch, ch)])

    return gather


def _score_body(flat_base, g_ref, adj_ref, tar_ref, add_ref, wlabel_ref,
                wsec_ref, w1_ref, w2_ref, l1wt_ref, l1b_ref, l2wt_ref,
                l2b_ref, l3w_ref, l3b_ref, out_ref):
    f32 = jnp.float32
    g = g_ref[...]
    sub = jnp.dot(jnp.dot(g, w1_ref[...], preferred_element_type=f32),
                  w2_ref[...], preferred_element_type=f32)
    concat = jnp.concatenate([
        jnp.broadcast_to(tar_ref[...], (_R, 64)),
        sub,
        jnp.broadcast_to(add_ref[...], (_R, 64)),
        adj_ref[...],
        jnp.broadcast_to(wlabel_ref[...], (_R, 256)),
        jnp.broadcast_to(wsec_ref[...], (_R, 256)),
    ], axis=1)
    h = jnp.dot(concat, l1wt_ref[...], preferred_element_type=f32) + l1b_ref[...]
    h = jnp.where(h >= 0, h, 0.01 * h)
    h = jnp.dot(h, l2wt_ref[...], preferred_element_type=f32) + l2b_ref[...]
    h = jnp.where(h >= 0, h, 0.01 * h)
    sc = lax.dot_general(l3w_ref[...], h, (((1,), (1,)), ((), ())),
                         preferred_element_type=f32) + l3b_ref[...]
    i = pl.program_id(0)
    flat = flat_base + i * _R + lax.broadcasted_iota(jnp.int32, (1, _R), 1)
    out_ref[...] = jnp.where(flat < _S, sc, _NEG).reshape(_R // 128, 128)


def _topk_body(score_ref, disc_ref, u_ref, e_ref):
    rows = _SPAD // 128
    flat = (lax.broadcasted_iota(jnp.int32, (rows, 128), 0) * 128
            + lax.broadcasted_iota(jnp.int32, (rows, 128), 1))
    big = jnp.int32(2**30)
    disc_ref[...] = jnp.zeros((rows, 128), jnp.float32)
    u_ref[...] = score_ref[...] / 0.01
    e_ref[...] = jnp.exp(u_ref[...] - jnp.max(u_ref[...]))

    def it(t, c):
        emax = jnp.max(e_ref[...])
        do_rescale = emax < 1e-20

        @pl.when(do_rescale)
        def _():
            disc_ref[...] = disc_ref[...] + e_ref[...] * c
            e_ref[...] = jnp.exp(u_ref[...] - jnp.max(u_ref[...]))

        c2 = jnp.where(do_rescale, jnp.float32(0.0), c)
        emax2 = jnp.where(do_rescale, jnp.float32(1.0), emax)
        c3 = c2 + 1.0 / jnp.sum(e_ref[...])
        sel = jnp.min(jnp.where(e_ref[...] == emax2, flat, big))
        issel = flat == sel
        disc_ref[...] = disc_ref[...] + jnp.where(issel, emax2 * c3, 0.0)
        e_ref[...] = jnp.where(issel, jnp.float32(0.0), e_ref[...])
        u_ref[...] = jnp.where(issel, jnp.float32(-1e30), u_ref[...])
        return c3

    c_end = lax.fori_loop(0, _BUDGET, it, jnp.float32(0.0))
    disc_ref[...] = disc_ref[...] + e_ref[...] * c_end


def _score_call(g, adj_pad, tar_xw, add_xw, wlabel, wsec,
                w1, w2, l1wt, l1b, l2wt, l2b, l3w, l3b, flat_base=0):
    const = lambda i: (0, 0)
    nrows = g.shape[0]
    return pl.pallas_call(
        functools.partial(_score_body, flat_base),
        grid=(nrows // _R,),
        in_specs=[
            pl.BlockSpec((_R, _D), lambda i: (i, 0)),
            pl.BlockSpec((_R, 1), lambda i: (i, 0)),
            pl.BlockSpec((1, 64), const),
            pl.BlockSpec((1, 64), const),
            pl.BlockSpec((1, 256), const),
            pl.BlockSpec((1, 256), const),
            pl.BlockSpec((256, 128), const),
            pl.BlockSpec((128, 64), const),
            pl.BlockSpec((705, 512), const),
            pl.BlockSpec((1, 512), const),
            pl.BlockSpec((512, 32), const),
            pl.BlockSpec((1, 32), const),
            pl.BlockSpec((1, 32), const),
            pl.BlockSpec((1, 1), const),
        ],
        out_specs=pl.BlockSpec((_R // 128, 128), lambda i: (i, 0)),
        out_shape=jax.ShapeDtypeStruct((nrows // 128, 128), jnp.float32),
    )(g, adj_pad, tar_xw, add_xw, wlabel, wsec, w1, w2, l1wt, l1b, l2wt, l2b,
      l3w, l3b)


def _topk_call(score2d):
    return pl.pallas_call(
        _topk_body,
        out_shape=jax.ShapeDtypeStruct((_SPAD // 128, 128), jnp.float32),
        scratch_shapes=[pltpu.VMEM((_SPAD // 128, 128), jnp.float32),
                        pltpu.VMEM((_SPAD // 128, 128), jnp.float32)],
    )(score2d)


def kernel(budget, target, sub_graph_nodes, new_feat, adj_tensor, wlabel, wsec,
           weight1, weight2, l1_w, l1_b, l2_w, l2_b, l3_w, l3_b):
    f32 = jnp.float32
    tar_xw = (jnp.take(new_feat, target, axis=0) @ weight1) @ weight2
    add_xw = (new_feat[-1:, :] @ weight1) @ weight2

    idx_pad = jnp.concatenate(
        [sub_graph_nodes.astype(jnp.int32), jnp.zeros((_SPAD - _S,), jnp.int32)])
    adj_pad = jnp.concatenate(
        [adj_tensor, jnp.zeros((_SPAD - _S, 1), f32)], axis=0)

    pr = _SPAD // _PHASES
    ch = 80
    weights = (tar_xw, add_xw, wlabel, wsec, weight1, weight2,
               l1_w.T, l1_b[None, :], l2_w.T, l2_b[None, :],
               l3_w, l3_b[None, :])
    gs = [_sc_gather(pr, ch)(
        idx_pad[p * pr:(p + 1) * pr].reshape(_NW, -1, ch), new_feat)
        for p in range(_PHASES)]
    ss = [_score_call(gs[p], adj_pad[p * pr:(p + 1) * pr], *weights,
                      flat_base=p * pr) for p in range(_PHASES)]
    disc2d = _topk_call(jnp.concatenate(ss, axis=0) if _PHASES > 1 else ss[0])
    discrete = disc2d.reshape(-1)[:_S]
    return (discrete, sub_graph_nodes[None, :])
